# baseline (speedup 1.0000x reference)
import functools

import jax
import jax.numpy as jnp
from jax import lax
from jax.experimental import pallas as pl
from jax.experimental.pallas import tpu as pltpu
from jax.experimental.pallas import tpu_sc as plsc

B = 4096
D = 256
E = 16
K = 4
DFF = 512
BLK = 512
NBLK = B // BLK

TILE = 256
NPAD = B * K + E * TILE
NTILES = NPAD // TILE
NW = 32
APW = (B * K) // NW
TPW = B // NW

_INTERPRET = False


def _dot(a, b):
    return jnp.dot(a, b, preferred_element_type=jnp.float32)


def _topk_gates(logits):
    l = logits
    iota = jax.lax.broadcasted_iota(jnp.int32, l.shape, 1)
    tvs, tis = [], []
    for _ in range(K):
        m = jnp.max(l, axis=1, keepdims=True)
        idx = jnp.min(jnp.where(l == m, iota, E), axis=1, keepdims=True)
        tvs.append(m)
        tis.append(idx)
        l = jnp.where(iota == idx, -jnp.inf, l)
    topv = jnp.concatenate(tvs, axis=1)
    topi = jnp.concatenate(tis, axis=1)
    g = jnp.exp(topv - topv[:, 0:1])
    gates = g / jnp.sum(g, axis=1, keepdims=True)
    comb = jnp.zeros_like(logits)
    ind = jnp.zeros_like(logits)
    for k in range(K):
        sel = iota == tis[k]
        comb = comb + jnp.where(sel, gates[:, k:k + 1], 0.0)
        ind = ind + jnp.where(sel, 1.0, 0.0)
    return topi, tis, gates, comb, ind


def _ranks(tis, ind, cnt_ref):
    @pl.when(pl.program_id(0) == 0)
    def _():
        cnt_ref[0, 0, :] = jnp.zeros((E,), jnp.float32)

    off = cnt_ref[0, 0, :][None, :]
    r_i = jax.lax.broadcasted_iota(jnp.int32, (128, 128), 0)
    c_i = jax.lax.broadcasted_iota(jnp.int32, (128, 128), 1)
    Lx = (r_i > c_i).astype(jnp.float32)
    parts = []
    o = off
    for bk in range(BLK // 128):
        Mb = ind[bk * 128:(bk + 1) * 128]
        parts.append(_dot(Lx, Mb) + o)
        o = o + jnp.sum(Mb, axis=0, keepdims=True)
    cnt_ref[0, 0, :] = o[0]
    Rm = jnp.concatenate(parts, axis=0)
    iota = jax.lax.broadcasted_iota(jnp.int32, (BLK, E), 1)
    rks = []
    for k in range(K):
        sel = (iota == tis[k]).astype(jnp.float32)
        rks.append(jnp.sum(sel * Rm, axis=1, keepdims=True))
    return jnp.concatenate(rks, axis=1).astype(jnp.int32)


def _importance(logits, imp_ref):
    mx = jnp.max(logits, axis=1, keepdims=True)
    ex = jnp.exp(logits - mx)
    sm = ex / jnp.sum(ex, axis=1, keepdims=True)

    @pl.when(pl.program_id(0) == 0)
    def _():
        imp_ref[0, 0, :] = jnp.zeros((E,), jnp.float32)

    imp_ref[0, 0, :] += jnp.sum(sm, axis=0)


def _router_outputs(x, logits, topi_ref, gates_ref, rank_ref, imp_ref, cnt_ref,
                    base_ref):
    topi, tis, gates, _, ind = _topk_gates(logits)
    _importance(logits, imp_ref)
    rank = _ranks(tis, ind, cnt_ref)
    topi_ref[...] = topi
    gates_ref[...] = gates
    rank_ref[...] = rank

    @pl.when(pl.program_id(0) == NBLK - 1)
    def _():
        cntf = cnt_ref[0, 0, :][None, :]
        padded = jnp.floor((cntf + (TILE - 1)) * (1.0 / TILE)) * TILE
        r_i = jax.lax.broadcasted_iota(jnp.int32, (E, E), 0)
        c_i = jax.lax.broadcasted_iota(jnp.int32, (E, E), 1)
        excl = jnp.sum(jnp.where(r_i > c_i, padded, 0.0), axis=1)
        base_ref[0, 0, :] = excl


def _embed_body(opi_ref, a_ref, b_ref, c_ref, opt_ref, Wp_ref, bp_ref,
                Wr_ref, br_ref,
                x_ref, topi_ref, gates_ref, rank_ref, imp_ref, cnt_ref, base_ref):
    op = opi_ref[0, 0, :][:, None]
    av = a_ref[0, 0, :][:, None]
    bv = b_ref[0, 0, :][:, None]
    cv = c_ref[0, 0, :][:, None]
    i8 = jax.lax.broadcasted_iota(jnp.int32, (BLK, 8), 1)
    abits = ((av >> i8) & 1).astype(jnp.float32)
    bbits = ((bv >> i8) & 1).astype(jnp.float32)
    cf = cv.astype(jnp.float32)
    op_emb = jnp.zeros((BLK, 32), jnp.float32)
    for j in range(8):
        op_emb = jnp.where(op == j, opt_ref[j:j + 1, :], op_emb)
    feat = jnp.concatenate(
        [op_emb, abits, bbits, cf, jnp.zeros((BLK, 128 - 49), jnp.float32)], axis=1)
    x = _dot(feat, Wp_ref[...]) + bp_ref[...]
    logits = _dot(x, Wr_ref[...]) + br_ref[...]
    x_ref[...] = x
    _router_outputs(x, logits, topi_ref, gates_ref, rank_ref, imp_ref, cnt_ref,
                    base_ref)


def _router_body(x_ref, p_ref, Wr_ref, br_ref,
                 x1_ref, topi_ref, gates_ref, rank_ref, imp_ref, cnt_ref, base_ref):
    x = x_ref[...] + p_ref[...]
    logits = _dot(x, Wr_ref[...]) + br_ref[...]
    x1_ref[...] = x
    _router_outputs(x, logits, topi_ref, gates_ref, rank_ref, imp_ref, cnt_ref,
                    base_ref)


def _head_body(x_ref, p_ref, H1_ref, bh1_ref, H2_ref, bh2_ref, res_ref):
    x = x_ref[...] + p_ref[...]
    h = jnp.maximum(_dot(x, H1_ref[...]) + bh1_ref[...], 0.0)
    z = _dot(h, H2_ref[...]) + bh2_ref[...]
    res_ref[...] = 1.0 / (1.0 + jnp.exp(-z))


def _ffn_body(te_ref, ba_ref, cn_ref, xs_ref, sg_ref,
              W1_ref, b1_ref, W2_ref, b2_ref, ys_ref):
    i = pl.program_id(0)
    te = te_ref[i]
    limit = ba_ref[te] + cn_ref[te]
    x = xs_ref[...]
    h = jnp.maximum(_dot(x, W1_ref[0]) + b1_ref[0], 0.0)
    y = _dot(h, W2_ref[0]) + b2_ref[0]
    rows = i * TILE + jax.lax.broadcasted_iota(jnp.int32, (TILE, 1), 0)
    ys_ref[...] = jnp.where(rows < limit, y * sg_ref[...], 0.0)


def _ffn_call(texp, base, cnt, xs, sgate2, W1l, b1l, W2l, b2l):
    grid_spec = pltpu.PrefetchScalarGridSpec(
        num_scalar_prefetch=3,
        grid=(NTILES,),
        in_specs=[
            pl.BlockSpec((TILE, D), lambda i, te, ba, cn: (i, 0)),
            pl.BlockSpec((TILE, 1), lambda i, te, ba, cn: (i, 0)),
            pl.BlockSpec((1, D, DFF), lambda i, te, ba, cn: (te[i], 0, 0)),
            pl.BlockSpec((1, 1, DFF), lambda i, te, ba, cn: (te[i], 0, 0)),
            pl.BlockSpec((1, DFF, D), lambda i, te, ba, cn: (te[i], 0, 0)),
            pl.BlockSpec((1, 1, D), lambda i, te, ba, cn: (te[i], 0, 0)),
        ],
        out_specs=pl.BlockSpec((TILE, D), lambda i, te, ba, cn: (i, 0)),
    )
    return pl.pallas_call(
        _ffn_body,
        grid_spec=grid_spec,
        out_shape=jax.ShapeDtypeStruct((NPAD, D), jnp.float32),
        interpret=_INTERPRET,
    )(texp, base, cnt, xs, sgate2,
      W1l, b1l.reshape(E, 1, DFF), W2l, b2l.reshape(E, 1, D))


def _slots_body(topi_ref, rank_ref, base_ref, slot_ref, texp_ref):
    base_row = base_ref[0, 0, :][None, :]
    iota = jax.lax.broadcasted_iota(jnp.int32, (B, E), 1)
    cols = []
    for k in range(K):
        tk = topi_ref[:, k:k + 1]
        sel = jnp.where(iota == tk, base_row, 0.0)
        cols.append(jnp.sum(sel, axis=1, keepdims=True).astype(jnp.int32))
    slot_ref[...] = jnp.concatenate(cols, axis=1) + rank_ref[...]
    ts = (base_row * (1.0 / TILE)).astype(jnp.int32)
    jv = jax.lax.broadcasted_iota(jnp.int32, (NTILES, E), 0)
    texp_ref[...] = (jnp.sum((jv >= ts).astype(jnp.int32), axis=1, keepdims=True)
                     - 1)


def _slots_call(topi, rank, bas):
    return pl.pallas_call(
        _slots_body,
        in_specs=[pl.BlockSpec((B, K), lambda: (0, 0)),
                  pl.BlockSpec((B, K), lambda: (0, 0)),
                  pl.BlockSpec((1, 1, E), lambda: (0, 0, 0))],
        out_specs=[pl.BlockSpec((B, K), lambda: (0, 0)),
                   pl.BlockSpec((NTILES, 1), lambda: (0, 0))],
        out_shape=[jax.ShapeDtypeStruct((B, K), jnp.int32),
                   jax.ShapeDtypeStruct((NTILES, 1), jnp.int32)],
        interpret=_INTERPRET,
    )(topi, rank, bas)


def _sc_load_slots(slot_hbm, slot_v, tok_v, abase):
    for j in range(4):
        pltpu.sync_copy(slot_hbm.at[pl.ds(abase + j * 128, 128)], slot_v.at[j])
    i16 = jax.lax.broadcasted_iota(jnp.int32, (16,), 0)
    for g in range(APW // 16):
        tok_v[g // 8, pl.ds((g % 8) * 16, 16)] = (abase + g * 16 + i16) >> 2


def _dispatch_body(slot_hbm, gates_hbm, x_hbm,
                   xs_hbm, sg_hbm,
                   g_v, slot_v, tok_v, rows_v, sem):
    wid = lax.axis_index("s") * 2 + lax.axis_index("c")
    abase = wid * APW
    pltpu.sync_copy(gates_hbm.at[pl.ds(abase, APW)], g_v)
    _sc_load_slots(slot_hbm, slot_v, tok_v, abase)
    for j in range(4):
        pltpu.async_copy(x_hbm.at[tok_v.at[j]], rows_v, sem).wait()
        pltpu.async_copy(rows_v, xs_hbm.at[slot_v.at[j]], sem).wait()
        pltpu.sync_copy(g_v.at[pl.ds(j * 128, 128)], sg_hbm.at[slot_v.at[j]])


def _combine_body(slot_hbm, ys_hbm,
                  p_hbm,
                  slot_v, rows_v, out_v, sem):
    wid = lax.axis_index("s") * 2 + lax.axis_index("c")
    abase = wid * APW
    tokbase = wid * TPW
    for j in range(4):
        pltpu.sync_copy(slot_hbm.at[pl.ds(abase + j * 128, 128)], slot_v.at[j])
    for j in range(4):
        pltpu.async_copy(ys_hbm.at[slot_v.at[j]], rows_v, sem).wait()

        def body(r, carry):
            for c in range(D // 16):
                sl = pl.ds(c * 16, 16)
                o = rows_v[4 * r, sl] + rows_v[4 * r + 1, sl]
                o = o + rows_v[4 * r + 2, sl]
                o = o + rows_v[4 * r + 3, sl]
                out_v[r, sl] = o
            return carry

        lax.fori_loop(0, 32, body, 0)
        pltpu.sync_copy(out_v, p_hbm.at[pl.ds(tokbase + j * 32, 32)])


def _sc_mesh():
    return plsc.VectorSubcoreMesh(core_axis_name="c", subcore_axis_name="s")


def _sc_dispatch(slot_flat, gates_flat, x):
    fn = functools.partial(
        pl.kernel,
        out_type=[
            jax.ShapeDtypeStruct((NPAD, D), jnp.float32),
            jax.ShapeDtypeStruct((NPAD,), jnp.float32),
        ],
        mesh=_sc_mesh(),
        scratch_types=[
            pltpu.VMEM((APW,), jnp.float32),
            pltpu.VMEM((4, 128), jnp.int32),
            pltpu.VMEM((4, 128), jnp.int32),
            pltpu.VMEM((128, D), jnp.float32),
            pltpu.SemaphoreType.DMA,
        ],
    )(_dispatch_body)
    return fn(slot_flat, gates_flat, x)


def _sc_combine(slot_flat, ys):
    fn = functools.partial(
        pl.kernel,
        out_type=[jax.ShapeDtypeStruct((B, D), jnp.float32)],
        mesh=_sc_mesh(),
        scratch_types=[
            pltpu.VMEM((4, 128), jnp.int32),
            pltpu.VMEM((128, D), jnp.float32),
            pltpu.VMEM((32, D), jnp.float32),
            pltpu.SemaphoreType.DMA,
        ],
    )(_combine_body)
    (p,) = fn(slot_flat, ys)
    return p


def _full(shape):
    nd = len(shape)
    return pl.BlockSpec(shape, lambda i: (0,) * nd)


_TOKSPEC = lambda: pl.BlockSpec((BLK, D), lambda i: (i, 0))
_K4SPEC_I = lambda: pl.BlockSpec((BLK, K), lambda i: (i, 0))
_ACCSPEC = lambda: pl.BlockSpec((1, 1, E), lambda i: (0, 0, 0))


def _router_outs():
    acc = jax.ShapeDtypeStruct((1, 1, E), jnp.float32)
    return [jax.ShapeDtypeStruct((B, D), jnp.float32),
            jax.ShapeDtypeStruct((B, K), jnp.int32),
            jax.ShapeDtypeStruct((B, K), jnp.float32),
            jax.ShapeDtypeStruct((B, K), jnp.int32),
            acc, acc, acc]


def _router_outspecs():
    return [_TOKSPEC(), _K4SPEC_I(), _K4SPEC_I(), _K4SPEC_I(),
            _ACCSPEC(), _ACCSPEC(), _ACCSPEC()]


def kernel(op_idx, a, b, c, op_table, Wp, bp, Wr, br, W1, b1, W2, b2, H1, bh1, H2, bh2):
    tok3 = lambda v: v.reshape(NBLK, 1, BLK)
    tokspec = pl.BlockSpec((1, 1, BLK), lambda i: (i, 0, 0))

    x0, topi0, gates0, rank0, imp0, cnt0, bas0 = pl.pallas_call(
        _embed_body,
        grid=(NBLK,),
        in_specs=[tokspec, tokspec, tokspec, tokspec,
                  _full((8, 32)), _full((128, D)), _full((1, D)),
                  _full((D, E)), _full((1, E))],
        out_specs=_router_outspecs(),
        out_shape=_router_outs(),
        interpret=_INTERPRET,
    )(tok3(op_idx), tok3(a), tok3(b), tok3(c), op_table,
      jnp.pad(Wp, ((0, 128 - 49), (0, 0))), bp[None, :], Wr[0], br[0][None, :])

    x_in, topi_l, imps, cnts = x0, topi0, [], []
    gates_l, rank_l, bas_l = gates0, rank0, bas0
    imps.append(imp0)
    cnts.append(cnt0)

    for l in range(2):
        cnt_i = cnts[l][0, 0].astype(jnp.int32)
        base_i = bas_l[0, 0].astype(jnp.int32)
        slot, texp2 = _slots_call(topi_l, rank_l, bas_l)
        slot_flat = slot.reshape(-1)
        texp = texp2.reshape(-1)
        xs, sgate = _sc_dispatch(slot_flat, gates_l.reshape(-1), x_in)
        ys = _ffn_call(texp, base_i, cnt_i, xs, sgate.reshape(NPAD, 1),
                       W1[l], b1[l], W2[l], b2[l])
        p = _sc_combine(slot_flat, ys)
        if l == 0:
            x1, topi1, gates1, rank1, imp1, cnt1, bas1 = pl.pallas_call(
                _router_body,
                grid=(NBLK,),
                in_specs=[_TOKSPEC(), _TOKSPEC(),
                          _full((D, E)), _full((1, E))],
                out_specs=_router_outspecs(),
                out_shape=_router_outs(),
                interpret=_INTERPRET,
            )(x_in, p, Wr[1], br[1][None, :])
            x_in, topi_l, gates_l, rank_l, bas_l = x1, topi1, gates1, rank1, bas1
            imps.append(imp1)
            cnts.append(cnt1)
        else:
            res = pl.pallas_call(
                _head_body,
                grid=(NBLK,),
                in_specs=[_TOKSPEC(), _TOKSPEC(),
                          _full((D, 64)), _full((1, 64)),
                          _full((64, 8)), _full((1, 8))],
                out_specs=pl.BlockSpec((BLK, 8), lambda i: (i, 0)),
                out_shape=jax.ShapeDtypeStruct((B, 8), jnp.float32),
                interpret=_INTERPRET,
            )(x_in, p, H1, bh1[None, :], H2, bh2[None, :])

    inv_b = 1.0 / B
    aux = jnp.float32(0.0)
    for l in range(2):
        aux = aux + E * jnp.sum(imps[l][0, 0] * inv_b * (cnts[l][0, 0] * inv_b))
    return (res, topi_l, aux)


# speedup vs baseline: 1.1115x; 1.1115x over previous
import functools

import jax
import jax.numpy as jnp
from jax import lax
from jax.experimental import pallas as pl
from jax.experimental.pallas import tpu as pltpu
from jax.experimental.pallas import tpu_sc as plsc

B = 4096
D = 256
E = 16
K = 4
DFF = 512
BLK = 512
NBLK = B // BLK

TILE = 256
NPAD = B * K + E * TILE
NTILES = NPAD // TILE
NW = 32
APW = (B * K) // NW
TPW = B // NW

_INTERPRET = False


def _dot(a, b):
    return jnp.dot(a, b, preferred_element_type=jnp.float32)


def _topk_gates(logits):
    l = logits
    iota = jax.lax.broadcasted_iota(jnp.int32, l.shape, 1)
    tvs, tis = [], []
    for _ in range(K):
        m = jnp.max(l, axis=1, keepdims=True)
        idx = jnp.min(jnp.where(l == m, iota, E), axis=1, keepdims=True)
        tvs.append(m)
        tis.append(idx)
        l = jnp.where(iota == idx, -jnp.inf, l)
    topv = jnp.concatenate(tvs, axis=1)
    topi = jnp.concatenate(tis, axis=1)
    g = jnp.exp(topv - topv[:, 0:1])
    gates = g / jnp.sum(g, axis=1, keepdims=True)
    comb = jnp.zeros_like(logits)
    ind = jnp.zeros_like(logits)
    for k in range(K):
        sel = iota == tis[k]
        comb = comb + jnp.where(sel, gates[:, k:k + 1], 0.0)
        ind = ind + jnp.where(sel, 1.0, 0.0)
    return topi, tis, gates, comb, ind


def _ranks(tis, ind, cnt_ref):
    @pl.when(pl.program_id(0) == 0)
    def _():
        cnt_ref[0, 0, :] = jnp.zeros((E,), jnp.float32)

    off = cnt_ref[0, 0, :][None, :]
    r_i = jax.lax.broadcasted_iota(jnp.int32, (128, 128), 0)
    c_i = jax.lax.broadcasted_iota(jnp.int32, (128, 128), 1)
    Lx = (r_i > c_i).astype(jnp.float32)
    parts = []
    o = off
    for bk in range(BLK // 128):
        Mb = ind[bk * 128:(bk + 1) * 128]
        parts.append(_dot(Lx, Mb) + o)
        o = o + jnp.sum(Mb, axis=0, keepdims=True)
    cnt_ref[0, 0, :] = o[0]
    Rm = jnp.concatenate(parts, axis=0)
    iota = jax.lax.broadcasted_iota(jnp.int32, (BLK, E), 1)
    rks = []
    for k in range(K):
        sel = (iota == tis[k]).astype(jnp.float32)
        rks.append(jnp.sum(sel * Rm, axis=1, keepdims=True))
    return jnp.concatenate(rks, axis=1).astype(jnp.int32)


def _importance(logits, imp_ref):
    mx = jnp.max(logits, axis=1, keepdims=True)
    ex = jnp.exp(logits - mx)
    sm = ex / jnp.sum(ex, axis=1, keepdims=True)

    @pl.when(pl.program_id(0) == 0)
    def _():
        imp_ref[0, 0, :] = jnp.zeros((E,), jnp.float32)

    imp_ref[0, 0, :] += jnp.sum(sm, axis=0)


def _router_outputs(x, logits, topi_ref, gates_ref, rank_ref, imp_ref, cnt_ref,
                    base_ref):
    topi, tis, gates, _, ind = _topk_gates(logits)
    _importance(logits, imp_ref)
    rank = _ranks(tis, ind, cnt_ref)
    topi_ref[...] = topi
    gates_ref[...] = gates
    rank_ref[...] = rank

    @pl.when(pl.program_id(0) == NBLK - 1)
    def _():
        cntf = cnt_ref[0, 0, :][None, :]
        padded = jnp.floor((cntf + (TILE - 1)) * (1.0 / TILE)) * TILE
        r_i = jax.lax.broadcasted_iota(jnp.int32, (E, E), 0)
        c_i = jax.lax.broadcasted_iota(jnp.int32, (E, E), 1)
        excl = jnp.sum(jnp.where(r_i > c_i, padded, 0.0), axis=1)
        base_ref[0, 0, :] = excl


def _embed_body(opi_ref, a_ref, b_ref, c_ref, opt_ref, Wp_ref, bp_ref,
                Wr_ref, br_ref,
                x_ref, topi_ref, gates_ref, rank_ref, imp_ref, cnt_ref, base_ref):
    op = opi_ref[0, 0, :][:, None]
    av = a_ref[0, 0, :][:, None]
    bv = b_ref[0, 0, :][:, None]
    cv = c_ref[0, 0, :][:, None]
    i8 = jax.lax.broadcasted_iota(jnp.int32, (BLK, 8), 1)
    abits = ((av >> i8) & 1).astype(jnp.float32)
    bbits = ((bv >> i8) & 1).astype(jnp.float32)
    cf = cv.astype(jnp.float32)
    op_emb = jnp.zeros((BLK, 32), jnp.float32)
    for j in range(8):
        op_emb = jnp.where(op == j, opt_ref[j:j + 1, :], op_emb)
    feat = jnp.concatenate(
        [op_emb, abits, bbits, cf, jnp.zeros((BLK, 128 - 49), jnp.float32)], axis=1)
    x = _dot(feat, Wp_ref[...]) + bp_ref[...]
    logits = _dot(x, Wr_ref[...]) + br_ref[...]
    x_ref[...] = x
    _router_outputs(x, logits, topi_ref, gates_ref, rank_ref, imp_ref, cnt_ref,
                    base_ref)


def _router_body(x_ref, p_ref, Wr_ref, br_ref,
                 x1_ref, topi_ref, gates_ref, rank_ref, imp_ref, cnt_ref, base_ref):
    x = x_ref[...] + p_ref[...]
    logits = _dot(x, Wr_ref[...]) + br_ref[...]
    x1_ref[...] = x
    _router_outputs(x, logits, topi_ref, gates_ref, rank_ref, imp_ref, cnt_ref,
                    base_ref)


def _head_body(x_ref, p_ref, H1_ref, bh1_ref, H2_ref, bh2_ref, res_ref):
    x = x_ref[...] + p_ref[...]
    h = jnp.maximum(_dot(x, H1_ref[...]) + bh1_ref[...], 0.0)
    z = _dot(h, H2_ref[...]) + bh2_ref[...]
    res_ref[...] = 1.0 / (1.0 + jnp.exp(-z))


def _ffn_body(te_ref, ba_ref, cn_ref, xs_ref, sg_ref,
              W1_ref, b1_ref, W2_ref, b2_ref, ys_ref):
    i = pl.program_id(0)
    te = te_ref[i]
    limit = ba_ref[te] + cn_ref[te]
    x = xs_ref[...]
    h = jnp.maximum(_dot(x, W1_ref[0]) + b1_ref[0], 0.0)
    y = _dot(h, W2_ref[0]) + b2_ref[0]
    rows = i * TILE + jax.lax.broadcasted_iota(jnp.int32, (TILE, 1), 0)
    ys_ref[...] = jnp.where(rows < limit, y * sg_ref[...], 0.0)


def _ffn_call(texp, base, cnt, xs, sgate2, W1l, b1l, W2l, b2l):
    grid_spec = pltpu.PrefetchScalarGridSpec(
        num_scalar_prefetch=3,
        grid=(NTILES,),
        in_specs=[
            pl.BlockSpec((TILE, D), lambda i, te, ba, cn: (i, 0)),
            pl.BlockSpec((TILE, 1), lambda i, te, ba, cn: (i, 0)),
            pl.BlockSpec((1, D, DFF), lambda i, te, ba, cn: (te[i], 0, 0)),
            pl.BlockSpec((1, 1, DFF), lambda i, te, ba, cn: (te[i], 0, 0)),
            pl.BlockSpec((1, DFF, D), lambda i, te, ba, cn: (te[i], 0, 0)),
            pl.BlockSpec((1, 1, D), lambda i, te, ba, cn: (te[i], 0, 0)),
        ],
        out_specs=pl.BlockSpec((TILE, D), lambda i, te, ba, cn: (i, 0)),
    )
    return pl.pallas_call(
        _ffn_body,
        grid_spec=grid_spec,
        out_shape=jax.ShapeDtypeStruct((NPAD, D), jnp.float32),
        interpret=_INTERPRET,
    )(texp, base, cnt, xs, sgate2,
      W1l, b1l.reshape(E, 1, DFF), W2l, b2l.reshape(E, 1, D))


def _slots_body(topi_ref, rank_ref, base_ref, slot_ref, texp_ref):
    base_row = base_ref[0, 0, :][None, :]
    iota = jax.lax.broadcasted_iota(jnp.int32, (B, E), 1)
    cols = []
    for k in range(K):
        tk = topi_ref[:, k:k + 1]
        sel = jnp.where(iota == tk, base_row, 0.0)
        cols.append(jnp.sum(sel, axis=1, keepdims=True).astype(jnp.int32))
    slot_ref[...] = jnp.concatenate(cols, axis=1) + rank_ref[...]
    ts = (base_row * (1.0 / TILE)).astype(jnp.int32)
    jv = jax.lax.broadcasted_iota(jnp.int32, (NTILES, E), 0)
    texp_ref[...] = (jnp.sum((jv >= ts).astype(jnp.int32), axis=1, keepdims=True)
                     - 1)


def _slots_call(topi, rank, bas):
    return pl.pallas_call(
        _slots_body,
        in_specs=[pl.BlockSpec((B, K), lambda: (0, 0)),
                  pl.BlockSpec((B, K), lambda: (0, 0)),
                  pl.BlockSpec((1, 1, E), lambda: (0, 0, 0))],
        out_specs=[pl.BlockSpec((B, K), lambda: (0, 0)),
                   pl.BlockSpec((NTILES, 1), lambda: (0, 0))],
        out_shape=[jax.ShapeDtypeStruct((B, K), jnp.int32),
                   jax.ShapeDtypeStruct((NTILES, 1), jnp.int32)],
        interpret=_INTERPRET,
    )(topi, rank, bas)


def _dispatch_body(slott_hbm, gatest_hbm, x_hbm,
                   xs_hbm, sg_hbm,
                   g_v, slot_v, rows_v, sem, sem2):
    wid = lax.axis_index("s") * 2 + lax.axis_index("c")
    tokbase = wid * TPW
    cp = pltpu.async_copy(x_hbm.at[pl.ds(tokbase, TPW)], rows_v, sem)
    for k in range(K):
        pltpu.sync_copy(slott_hbm.at[pl.ds(k * B + tokbase, TPW)], slot_v.at[k])
        pltpu.sync_copy(gatest_hbm.at[pl.ds(k * B + tokbase, TPW)], g_v.at[k])
    cp.wait()
    handles = []
    for k in range(K):
        handles.append(pltpu.async_copy(rows_v, xs_hbm.at[slot_v.at[k]], sem2))
        handles.append(pltpu.async_copy(g_v.at[k], sg_hbm.at[slot_v.at[k]], sem2))
    for h in handles:
        h.wait()


def _acc_rows(out_v, buf, first):
    def body(r, carry):
        for c in range(D // 16):
            sl = pl.ds(c * 16, 16)
            if first:
                out_v[r, sl] = buf[r, sl]
            else:
                out_v[r, sl] = out_v[r, sl] + buf[r, sl]
        return carry

    lax.fori_loop(0, TPW, body, 0)


def _combine_body(slott_hbm, ys_hbm,
                  p_hbm,
                  slot_v, bufa, bufb, out_v, sem0, sema, semb):
    wid = lax.axis_index("s") * 2 + lax.axis_index("c")
    tokbase = wid * TPW
    for k in range(K):
        pltpu.sync_copy(slott_hbm.at[pl.ds(k * B + tokbase, TPW)], slot_v.at[k])
    h0 = pltpu.async_copy(ys_hbm.at[slot_v.at[0]], out_v, sem0)
    h1 = pltpu.async_copy(ys_hbm.at[slot_v.at[1]], bufa, sema)
    h0.wait()
    h1.wait()
    h2 = pltpu.async_copy(ys_hbm.at[slot_v.at[2]], bufb, semb)
    _acc_rows(out_v, bufa, False)
    h2.wait()
    h3 = pltpu.async_copy(ys_hbm.at[slot_v.at[3]], bufa, sema)
    _acc_rows(out_v, bufb, False)
    h3.wait()
    _acc_rows(out_v, bufa, False)
    pltpu.sync_copy(out_v, p_hbm.at[pl.ds(tokbase, TPW)])


def _sc_mesh():
    return plsc.VectorSubcoreMesh(core_axis_name="c", subcore_axis_name="s")


def _sc_dispatch(slott_flat, gatest_flat, x):
    fn = functools.partial(
        pl.kernel,
        out_type=[
            jax.ShapeDtypeStruct((NPAD, D), jnp.float32),
            jax.ShapeDtypeStruct((NPAD,), jnp.float32),
        ],
        mesh=_sc_mesh(),
        scratch_types=[
            pltpu.VMEM((K, TPW), jnp.float32),
            pltpu.VMEM((K, TPW), jnp.int32),
            pltpu.VMEM((TPW, D), jnp.float32),
            pltpu.SemaphoreType.DMA,
            pltpu.SemaphoreType.DMA,
        ],
    )(_dispatch_body)
    return fn(slott_flat, gatest_flat, x)


def _sc_combine(slott_flat, ys):
    fn = functools.partial(
        pl.kernel,
        out_type=[jax.ShapeDtypeStruct((B, D), jnp.float32)],
        mesh=_sc_mesh(),
        scratch_types=[
            pltpu.VMEM((K, TPW), jnp.int32),
            pltpu.VMEM((TPW, D), jnp.float32),
            pltpu.VMEM((TPW, D), jnp.float32),
            pltpu.VMEM((TPW, D), jnp.float32),
            pltpu.SemaphoreType.DMA,
            pltpu.SemaphoreType.DMA,
            pltpu.SemaphoreType.DMA,
        ],
    )(_combine_body)
    (p,) = fn(slott_flat, ys)
    return p


def _full(shape):
    nd = len(shape)
    return pl.BlockSpec(shape, lambda i: (0,) * nd)


_TOKSPEC = lambda: pl.BlockSpec((BLK, D), lambda i: (i, 0))
_K4SPEC_I = lambda: pl.BlockSpec((BLK, K), lambda i: (i, 0))
_ACCSPEC = lambda: pl.BlockSpec((1, 1, E), lambda i: (0, 0, 0))


def _router_outs():
    acc = jax.ShapeDtypeStruct((1, 1, E), jnp.float32)
    return [jax.ShapeDtypeStruct((B, D), jnp.float32),
            jax.ShapeDtypeStruct((B, K), jnp.int32),
            jax.ShapeDtypeStruct((B, K), jnp.float32),
            jax.ShapeDtypeStruct((B, K), jnp.int32),
            acc, acc, acc]


def _router_outspecs():
    return [_TOKSPEC(), _K4SPEC_I(), _K4SPEC_I(), _K4SPEC_I(),
            _ACCSPEC(), _ACCSPEC(), _ACCSPEC()]


def kernel(op_idx, a, b, c, op_table, Wp, bp, Wr, br, W1, b1, W2, b2, H1, bh1, H2, bh2):
    tok3 = lambda v: v.reshape(NBLK, 1, BLK)
    tokspec = pl.BlockSpec((1, 1, BLK), lambda i: (i, 0, 0))

    x0, topi0, gates0, rank0, imp0, cnt0, bas0 = pl.pallas_call(
        _embed_body,
        grid=(NBLK,),
        in_specs=[tokspec, tokspec, tokspec, tokspec,
                  _full((8, 32)), _full((128, D)), _full((1, D)),
                  _full((D, E)), _full((1, E))],
        out_specs=_router_outspecs(),
        out_shape=_router_outs(),
        interpret=_INTERPRET,
    )(tok3(op_idx), tok3(a), tok3(b), tok3(c), op_table,
      jnp.pad(Wp, ((0, 128 - 49), (0, 0))), bp[None, :], Wr[0], br[0][None, :])

    x_in, topi_l, imps, cnts = x0, topi0, [], []
    gates_l, rank_l, bas_l = gates0, rank0, bas0
    imps.append(imp0)
    cnts.append(cnt0)

    for l in range(2):
        cnt_i = cnts[l][0, 0].astype(jnp.int32)
        base_i = bas_l[0, 0].astype(jnp.int32)
        slot, texp2 = _slots_call(topi_l, rank_l, bas_l)
        slott_flat = slot.T.reshape(-1)
        texp = texp2.reshape(-1)
        xs, sgate = _sc_dispatch(slott_flat, gates_l.T.reshape(-1), x_in)
        ys = _ffn_call(texp, base_i, cnt_i, xs, sgate.reshape(NPAD, 1),
                       W1[l], b1[l], W2[l], b2[l])
        p = _sc_combine(slott_flat, ys)
        if l == 0:
            x1, topi1, gates1, rank1, imp1, cnt1, bas1 = pl.pallas_call(
                _router_body,
                grid=(NBLK,),
                in_specs=[_TOKSPEC(), _TOKSPEC(),
                          _full((D, E)), _full((1, E))],
                out_specs=_router_outspecs(),
                out_shape=_router_outs(),
                interpret=_INTERPRET,
            )(x_in, p, Wr[1], br[1][None, :])
            x_in, topi_l, gates_l, rank_l, bas_l = x1, topi1, gates1, rank1, bas1
            imps.append(imp1)
            cnts.append(cnt1)
        else:
            res = pl.pallas_call(
                _head_body,
                grid=(NBLK,),
                in_specs=[_TOKSPEC(), _TOKSPEC(),
                          _full((D, 64)), _full((1, 64)),
                          _full((64, 8)), _full((1, 8))],
                out_specs=pl.BlockSpec((BLK, 8), lambda i: (i, 0)),
                out_shape=jax.ShapeDtypeStruct((B, 8), jnp.float32),
                interpret=_INTERPRET,
            )(x_in, p, H1, bh1[None, :], H2, bh2[None, :])

    inv_b = 1.0 / B
    aux = jnp.float32(0.0)
    for l in range(2):
        aux = aux + E * jnp.sum(imps[l][0, 0] * inv_b * (cnts[l][0, 0] * inv_b))
    return (res, topi_l, aux)


# speedup vs baseline: 1.4405x; 1.2960x over previous
import functools

import jax
import jax.numpy as jnp
from jax import lax
from jax.experimental import pallas as pl
from jax.experimental.pallas import tpu as pltpu
from jax.experimental.pallas import tpu_sc as plsc

B = 4096
D = 256
E = 16
K = 4
DFF = 512
BLK = 512
NBLK = B // BLK

TILE = 256
NPAD = B * K + E * TILE
NTILES = NPAD // TILE
NW = 32
APW = (B * K) // NW
TPW = B // NW

_INTERPRET = False


def _dot(a, b):
    return jnp.dot(a, b, preferred_element_type=jnp.float32)


def _topk_gates(logits):
    l = logits
    iota = jax.lax.broadcasted_iota(jnp.int32, l.shape, 1)
    tvs, tis = [], []
    for _ in range(K):
        m = jnp.max(l, axis=1, keepdims=True)
        idx = jnp.min(jnp.where(l == m, iota, E), axis=1, keepdims=True)
        tvs.append(m)
        tis.append(idx)
        l = jnp.where(iota == idx, -jnp.inf, l)
    topv = jnp.concatenate(tvs, axis=1)
    topi = jnp.concatenate(tis, axis=1)
    g = jnp.exp(topv - topv[:, 0:1])
    gates = g / jnp.sum(g, axis=1, keepdims=True)
    comb = jnp.zeros_like(logits)
    ind = jnp.zeros_like(logits)
    for k in range(K):
        sel = iota == tis[k]
        comb = comb + jnp.where(sel, gates[:, k:k + 1], 0.0)
        ind = ind + jnp.where(sel, 1.0, 0.0)
    return topi, tis, gates, comb, ind


def _ranks(tis, ind, cnt_ref):
    @pl.when(pl.program_id(0) == 0)
    def _():
        cnt_ref[0, 0, :] = jnp.zeros((E,), jnp.float32)

    off = cnt_ref[0, 0, :][None, :]
    r_i = jax.lax.broadcasted_iota(jnp.int32, (128, 128), 0)
    c_i = jax.lax.broadcasted_iota(jnp.int32, (128, 128), 1)
    Lx = (r_i > c_i).astype(jnp.float32)
    parts = []
    o = off
    for bk in range(BLK // 128):
        Mb = ind[bk * 128:(bk + 1) * 128]
        parts.append(_dot(Lx, Mb) + o)
        o = o + jnp.sum(Mb, axis=0, keepdims=True)
    cnt_ref[0, 0, :] = o[0]
    Rm = jnp.concatenate(parts, axis=0)
    iota = jax.lax.broadcasted_iota(jnp.int32, (BLK, E), 1)
    rks = []
    for k in range(K):
        sel = (iota == tis[k]).astype(jnp.float32)
        rks.append(jnp.sum(sel * Rm, axis=1, keepdims=True))
    return jnp.concatenate(rks, axis=1).astype(jnp.int32)


def _importance(logits, imp_ref):
    mx = jnp.max(logits, axis=1, keepdims=True)
    ex = jnp.exp(logits - mx)
    sm = ex / jnp.sum(ex, axis=1, keepdims=True)

    @pl.when(pl.program_id(0) == 0)
    def _():
        imp_ref[0, 0, :] = jnp.zeros((E,), jnp.float32)

    imp_ref[0, 0, :] += jnp.sum(sm, axis=0)


def _router_outputs(x, logits, topi_ref, gates_ref, rank_ref, imp_ref, cnt_ref,
                    base_ref):
    topi, tis, gates, _, ind = _topk_gates(logits)
    _importance(logits, imp_ref)
    rank = _ranks(tis, ind, cnt_ref)
    topi_ref[...] = topi
    gates_ref[...] = gates
    rank_ref[...] = rank

    @pl.when(pl.program_id(0) == NBLK - 1)
    def _():
        cntf = cnt_ref[0, 0, :][None, :]
        padded = jnp.floor((cntf + (TILE - 1)) * (1.0 / TILE)) * TILE
        r_i = jax.lax.broadcasted_iota(jnp.int32, (E, E), 0)
        c_i = jax.lax.broadcasted_iota(jnp.int32, (E, E), 1)
        excl = jnp.sum(jnp.where(r_i > c_i, padded, 0.0), axis=1)
        base_ref[0, 0, :] = excl


def _embed_body(opi_ref, a_ref, b_ref, c_ref, opt_ref, Wp_ref, bp_ref,
                Wr_ref, br_ref,
                x_ref, topi_ref, gates_ref, rank_ref, imp_ref, cnt_ref, base_ref):
    op = opi_ref[0, 0, :][:, None]
    av = a_ref[0, 0, :][:, None]
    bv = b_ref[0, 0, :][:, None]
    cv = c_ref[0, 0, :][:, None]
    i8 = jax.lax.broadcasted_iota(jnp.int32, (BLK, 8), 1)
    abits = ((av >> i8) & 1).astype(jnp.float32)
    bbits = ((bv >> i8) & 1).astype(jnp.float32)
    cf = cv.astype(jnp.float32)
    op_emb = jnp.zeros((BLK, 32), jnp.float32)
    for j in range(8):
        op_emb = jnp.where(op == j, opt_ref[j:j + 1, :], op_emb)
    feat = jnp.concatenate(
        [op_emb, abits, bbits, cf, jnp.zeros((BLK, 128 - 49), jnp.float32)], axis=1)
    x = _dot(feat, Wp_ref[...]) + bp_ref[...]
    logits = _dot(x, Wr_ref[...]) + br_ref[...]
    x_ref[...] = x
    _router_outputs(x, logits, topi_ref, gates_ref, rank_ref, imp_ref, cnt_ref,
                    base_ref)


def _combine_gated(x_ref, c0_ref, c1_ref, c2_ref, c3_ref, g_ref):
    g = g_ref[...]
    out = g[:, 0:1] * c0_ref[...]
    out = out + g[:, 1:2] * c1_ref[...]
    out = out + g[:, 2:3] * c2_ref[...]
    out = out + g[:, 3:4] * c3_ref[...]
    return x_ref[...] + out


def _router_body(x_ref, c0_ref, c1_ref, c2_ref, c3_ref, g_ref, Wr_ref, br_ref,
                 x1_ref, topi_ref, gates_ref, rank_ref, imp_ref, cnt_ref, base_ref):
    x = _combine_gated(x_ref, c0_ref, c1_ref, c2_ref, c3_ref, g_ref)
    logits = _dot(x, Wr_ref[...]) + br_ref[...]
    x1_ref[...] = x
    _router_outputs(x, logits, topi_ref, gates_ref, rank_ref, imp_ref, cnt_ref,
                    base_ref)


def _head_body(x_ref, c0_ref, c1_ref, c2_ref, c3_ref, g_ref,
               H1_ref, bh1_ref, H2_ref, bh2_ref, res_ref):
    x = _combine_gated(x_ref, c0_ref, c1_ref, c2_ref, c3_ref, g_ref)
    h = jnp.maximum(_dot(x, H1_ref[...]) + bh1_ref[...], 0.0)
    z = _dot(h, H2_ref[...]) + bh2_ref[...]
    res_ref[...] = 1.0 / (1.0 + jnp.exp(-z))


def _ffn_body(te_ref, xs_ref, W1_ref, b1_ref, W2_ref, b2_ref, ys_ref):
    x = xs_ref[...]
    h = jnp.maximum(_dot(x, W1_ref[0]) + b1_ref[0], 0.0)
    ys_ref[...] = _dot(h, W2_ref[0]) + b2_ref[0]


def _ffn_call(texp, xs, W1l, b1l, W2l, b2l):
    grid_spec = pltpu.PrefetchScalarGridSpec(
        num_scalar_prefetch=1,
        grid=(NTILES,),
        in_specs=[
            pl.BlockSpec((TILE, D), lambda i, te: (i, 0)),
            pl.BlockSpec((1, D, DFF), lambda i, te: (te[i], 0, 0)),
            pl.BlockSpec((1, 1, DFF), lambda i, te: (te[i], 0, 0)),
            pl.BlockSpec((1, DFF, D), lambda i, te: (te[i], 0, 0)),
            pl.BlockSpec((1, 1, D), lambda i, te: (te[i], 0, 0)),
        ],
        out_specs=pl.BlockSpec((TILE, D), lambda i, te: (i, 0)),
    )
    return pl.pallas_call(
        _ffn_body,
        grid_spec=grid_spec,
        out_shape=jax.ShapeDtypeStruct((NPAD, D), jnp.float32),
        interpret=_INTERPRET,
    )(texp, xs, W1l, b1l.reshape(E, 1, DFF), W2l, b2l.reshape(E, 1, D))


def _slots_body(topi_ref, rank_ref, base_ref, slot_ref, texp_ref):
    base_row = base_ref[0, 0, :][None, :]
    iota = jax.lax.broadcasted_iota(jnp.int32, (B, E), 1)
    cols = []
    for k in range(K):
        tk = topi_ref[:, k:k + 1]
        sel = jnp.where(iota == tk, base_row, 0.0)
        cols.append(jnp.sum(sel, axis=1, keepdims=True).astype(jnp.int32))
    slot_ref[...] = jnp.concatenate(cols, axis=1) + rank_ref[...]
    ts = (base_row * (1.0 / TILE)).astype(jnp.int32)
    jv = jax.lax.broadcasted_iota(jnp.int32, (NTILES, E), 0)
    texp_ref[...] = (jnp.sum((jv >= ts).astype(jnp.int32), axis=1, keepdims=True)
                     - 1)


def _slots_call(topi, rank, bas):
    return pl.pallas_call(
        _slots_body,
        in_specs=[pl.BlockSpec((B, K), lambda: (0, 0)),
                  pl.BlockSpec((B, K), lambda: (0, 0)),
                  pl.BlockSpec((1, 1, E), lambda: (0, 0, 0))],
        out_specs=[pl.BlockSpec((B, K), lambda: (0, 0)),
                   pl.BlockSpec((NTILES, 1), lambda: (0, 0))],
        out_shape=[jax.ShapeDtypeStruct((B, K), jnp.int32),
                   jax.ShapeDtypeStruct((NTILES, 1), jnp.int32)],
        interpret=_INTERPRET,
    )(topi, rank, bas)


def _dispatch_body(slott_hbm, x_hbm,
                   xs_hbm,
                   slot_v, rows_v, sem, sem2):
    wid = lax.axis_index("s") * 2 + lax.axis_index("c")
    tokbase = wid * TPW
    cp = pltpu.async_copy(x_hbm.at[pl.ds(tokbase, TPW)], rows_v, sem)
    for k in range(K):
        pltpu.sync_copy(slott_hbm.at[pl.ds(k * B + tokbase, TPW)], slot_v.at[k])
    cp.wait()
    handles = [pltpu.async_copy(rows_v, xs_hbm.at[slot_v.at[k]], sem2)
               for k in range(K)]
    for h in handles:
        h.wait()


def _combine_body(slott_hbm, ys_hbm,
                  c_hbm,
                  slot_v, bufa, bufb, semga, semgb, semwa, semwb):
    wid = lax.axis_index("s") * 2 + lax.axis_index("c")
    tokbase = wid * TPW
    for k in range(K):
        pltpu.sync_copy(slott_hbm.at[pl.ds(k * B + tokbase, TPW)], slot_v.at[k])
    g0 = pltpu.async_copy(ys_hbm.at[slot_v.at[0]], bufa, semga)
    g1 = pltpu.async_copy(ys_hbm.at[slot_v.at[1]], bufb, semgb)
    g0.wait()
    w0 = pltpu.async_copy(bufa, c_hbm.at[0, pl.ds(tokbase, TPW)], semwa)
    g1.wait()
    w1 = pltpu.async_copy(bufb, c_hbm.at[1, pl.ds(tokbase, TPW)], semwb)
    w0.wait()
    g2 = pltpu.async_copy(ys_hbm.at[slot_v.at[2]], bufa, semga)
    w1.wait()
    g3 = pltpu.async_copy(ys_hbm.at[slot_v.at[3]], bufb, semgb)
    g2.wait()
    w2 = pltpu.async_copy(bufa, c_hbm.at[2, pl.ds(tokbase, TPW)], semwa)
    g3.wait()
    w3 = pltpu.async_copy(bufb, c_hbm.at[3, pl.ds(tokbase, TPW)], semwb)
    w2.wait()
    w3.wait()


def _sc_mesh():
    return plsc.VectorSubcoreMesh(core_axis_name="c", subcore_axis_name="s")


def _sc_dispatch(slott_flat, x):
    fn = functools.partial(
        pl.kernel,
        out_type=[jax.ShapeDtypeStruct((NPAD, D), jnp.float32)],
        mesh=_sc_mesh(),
        scratch_types=[
            pltpu.VMEM((K, TPW), jnp.int32),
            pltpu.VMEM((TPW, D), jnp.float32),
            pltpu.SemaphoreType.DMA,
            pltpu.SemaphoreType.DMA,
        ],
    )(_dispatch_body)
    (xs,) = fn(slott_flat, x)
    return xs


def _sc_combine(slott_flat, ys):
    fn = functools.partial(
        pl.kernel,
        out_type=[jax.ShapeDtypeStruct((K, B, D), jnp.float32)],
        mesh=_sc_mesh(),
        scratch_types=[
            pltpu.VMEM((K, TPW), jnp.int32),
            pltpu.VMEM((TPW, D), jnp.float32),
            pltpu.VMEM((TPW, D), jnp.float32),
            pltpu.SemaphoreType.DMA,
            pltpu.SemaphoreType.DMA,
            pltpu.SemaphoreType.DMA,
            pltpu.SemaphoreType.DMA,
        ],
    )(_combine_body)
    (c,) = fn(slott_flat, ys)
    return c


def _full(shape):
    nd = len(shape)
    return pl.BlockSpec(shape, lambda i: (0,) * nd)


_TOKSPEC = lambda: pl.BlockSpec((BLK, D), lambda i: (i, 0))
_K4SPEC_I = lambda: pl.BlockSpec((BLK, K), lambda i: (i, 0))
_ACCSPEC = lambda: pl.BlockSpec((1, 1, E), lambda i: (0, 0, 0))


def _router_outs():
    acc = jax.ShapeDtypeStruct((1, 1, E), jnp.float32)
    return [jax.ShapeDtypeStruct((B, D), jnp.float32),
            jax.ShapeDtypeStruct((B, K), jnp.int32),
            jax.ShapeDtypeStruct((B, K), jnp.float32),
            jax.ShapeDtypeStruct((B, K), jnp.int32),
            acc, acc, acc]


def _router_outspecs():
    return [_TOKSPEC(), _K4SPEC_I(), _K4SPEC_I(), _K4SPEC_I(),
            _ACCSPEC(), _ACCSPEC(), _ACCSPEC()]


def kernel(op_idx, a, b, c, op_table, Wp, bp, Wr, br, W1, b1, W2, b2, H1, bh1, H2, bh2):
    tok3 = lambda v: v.reshape(NBLK, 1, BLK)
    tokspec = pl.BlockSpec((1, 1, BLK), lambda i: (i, 0, 0))

    x0, topi0, gates0, rank0, imp0, cnt0, bas0 = pl.pallas_call(
        _embed_body,
        grid=(NBLK,),
        in_specs=[tokspec, tokspec, tokspec, tokspec,
                  _full((8, 32)), _full((128, D)), _full((1, D)),
                  _full((D, E)), _full((1, E))],
        out_specs=_router_outspecs(),
        out_shape=_router_outs(),
        interpret=_INTERPRET,
    )(tok3(op_idx), tok3(a), tok3(b), tok3(c), op_table,
      jnp.pad(Wp, ((0, 128 - 49), (0, 0))), bp[None, :], Wr[0], br[0][None, :])

    x_in, topi_l, imps, cnts = x0, topi0, [], []
    gates_l, rank_l, bas_l = gates0, rank0, bas0
    imps.append(imp0)
    cnts.append(cnt0)

    for l in range(2):
        slot, texp2 = _slots_call(topi_l, rank_l, bas_l)
        slott_flat = slot.T.reshape(-1)
        texp = texp2.reshape(-1)
        xs = _sc_dispatch(slott_flat, x_in)
        ys = _ffn_call(texp, xs, W1[l], b1[l], W2[l], b2[l])
        c = _sc_combine(slott_flat, ys)
        cin = [c[0], c[1], c[2], c[3], gates_l]
        cspecs = [_TOKSPEC(), _TOKSPEC(), _TOKSPEC(), _TOKSPEC(), _K4SPEC_I()]
        if l == 0:
            x1, topi1, gates1, rank1, imp1, cnt1, bas1 = pl.pallas_call(
                _router_body,
                grid=(NBLK,),
                in_specs=[_TOKSPEC()] + cspecs + [_full((D, E)), _full((1, E))],
                out_specs=_router_outspecs(),
                out_shape=_router_outs(),
                interpret=_INTERPRET,
            )(x_in, *cin, Wr[1], br[1][None, :])
            x_in, topi_l, gates_l, rank_l, bas_l = x1, topi1, gates1, rank1, bas1
            imps.append(imp1)
            cnts.append(cnt1)
        else:
            res = pl.pallas_call(
                _head_body,
                grid=(NBLK,),
                in_specs=[_TOKSPEC()] + cspecs +
                         [_full((D, 64)), _full((1, 64)),
                          _full((64, 8)), _full((1, 8))],
                out_specs=pl.BlockSpec((BLK, 8), lambda i: (i, 0)),
                out_shape=jax.ShapeDtypeStruct((B, 8), jnp.float32),
                interpret=_INTERPRET,
            )(x_in, *cin, H1, bh1[None, :], H2, bh2[None, :])

    inv_b = 1.0 / B
    aux = jnp.float32(0.0)
    for l in range(2):
        aux = aux + E * jnp.sum(imps[l][0, 0] * inv_b * (cnts[l][0, 0] * inv_b))
    return (res, topi_l, aux)


# speedup vs baseline: 1.5427x; 1.0709x over previous
import functools

import jax
import jax.numpy as jnp
from jax import lax
from jax.experimental import pallas as pl
from jax.experimental.pallas import tpu as pltpu
from jax.experimental.pallas import tpu_sc as plsc

B = 4096
D = 256
E = 16
K = 4
DFF = 512
BLK = 512
NBLK = B // BLK

TILE = 256
NPAD = B * K + E * TILE
NTILES = NPAD // TILE
NW = 32
APW = (B * K) // NW
TPW = B // NW

_INTERPRET = False


def _dot(a, b):
    return jnp.dot(a, b, preferred_element_type=jnp.float32)


def _topk_gates(logits):
    l = logits
    iota = jax.lax.broadcasted_iota(jnp.int32, l.shape, 1)
    tvs, tis = [], []
    for _ in range(K):
        m = jnp.max(l, axis=1, keepdims=True)
        idx = jnp.min(jnp.where(l == m, iota, E), axis=1, keepdims=True)
        tvs.append(m)
        tis.append(idx)
        l = jnp.where(iota == idx, -jnp.inf, l)
    topv = jnp.concatenate(tvs, axis=1)
    topi = jnp.concatenate(tis, axis=1)
    g = jnp.exp(topv - topv[:, 0:1])
    gates = g / jnp.sum(g, axis=1, keepdims=True)
    comb = jnp.zeros_like(logits)
    ind = jnp.zeros_like(logits)
    for k in range(K):
        sel = iota == tis[k]
        comb = comb + jnp.where(sel, gates[:, k:k + 1], 0.0)
        ind = ind + jnp.where(sel, 1.0, 0.0)
    return topi, tis, gates, comb, ind


def _ranks(tis, ind, cnt_ref):
    @pl.when(pl.program_id(0) == 0)
    def _():
        cnt_ref[0, 0, :] = jnp.zeros((E,), jnp.float32)

    off = cnt_ref[0, 0, :][None, :]
    r_i = jax.lax.broadcasted_iota(jnp.int32, (128, 128), 0)
    c_i = jax.lax.broadcasted_iota(jnp.int32, (128, 128), 1)
    Lx = (r_i > c_i).astype(jnp.float32)
    parts = []
    o = off
    for bk in range(BLK // 128):
        Mb = ind[bk * 128:(bk + 1) * 128]
        parts.append(_dot(Lx, Mb) + o)
        o = o + jnp.sum(Mb, axis=0, keepdims=True)
    cnt_ref[0, 0, :] = o[0]
    Rm = jnp.concatenate(parts, axis=0)
    iota = jax.lax.broadcasted_iota(jnp.int32, (BLK, E), 1)
    rks = []
    for k in range(K):
        sel = (iota == tis[k]).astype(jnp.float32)
        rks.append(jnp.sum(sel * Rm, axis=1, keepdims=True))
    return jnp.concatenate(rks, axis=1).astype(jnp.int32)


def _importance(logits, imp_ref):
    mx = jnp.max(logits, axis=1, keepdims=True)
    ex = jnp.exp(logits - mx)
    sm = ex / jnp.sum(ex, axis=1, keepdims=True)

    @pl.when(pl.program_id(0) == 0)
    def _():
        imp_ref[0, 0, :] = jnp.zeros((E,), jnp.float32)

    imp_ref[0, 0, :] += jnp.sum(sm, axis=0)


def _router_outputs(x, logits, topi_ref, gates_ref, rank_ref, imp_ref, cnt_ref,
                    base_ref):
    topi, tis, gates, _, ind = _topk_gates(logits)
    _importance(logits, imp_ref)
    rank = _ranks(tis, ind, cnt_ref)
    topi_ref[...] = topi
    gates_ref[...] = gates
    rank_ref[...] = rank

    @pl.when(pl.program_id(0) == NBLK - 1)
    def _():
        cntf = cnt_ref[0, 0, :][None, :]
        padded = jnp.floor((cntf + (TILE - 1)) * (1.0 / TILE)) * TILE
        r_i = jax.lax.broadcasted_iota(jnp.int32, (E, E), 0)
        c_i = jax.lax.broadcasted_iota(jnp.int32, (E, E), 1)
        excl = jnp.sum(jnp.where(r_i > c_i, padded, 0.0), axis=1)
        base_ref[0, 0, :] = excl


def _embed_body(opi_ref, a_ref, b_ref, c_ref, opt_ref, Wp_ref, bp_ref,
                Wr_ref, br_ref,
                x_ref, topi_ref, gates_ref, rank_ref, imp_ref, cnt_ref, base_ref):
    op = opi_ref[0, 0, :][:, None]
    av = a_ref[0, 0, :][:, None]
    bv = b_ref[0, 0, :][:, None]
    cv = c_ref[0, 0, :][:, None]
    i8 = jax.lax.broadcasted_iota(jnp.int32, (BLK, 8), 1)
    abits = ((av >> i8) & 1).astype(jnp.float32)
    bbits = ((bv >> i8) & 1).astype(jnp.float32)
    cf = cv.astype(jnp.float32)
    op_emb = jnp.zeros((BLK, 32), jnp.float32)
    for j in range(8):
        op_emb = jnp.where(op == j, opt_ref[j:j + 1, :], op_emb)
    feat = jnp.concatenate(
        [op_emb, abits, bbits, cf, jnp.zeros((BLK, 128 - 49), jnp.float32)], axis=1)
    x = _dot(feat, Wp_ref[...]) + bp_ref[...]
    logits = _dot(x, Wr_ref[...]) + br_ref[...]
    x_ref[...] = x
    _router_outputs(x, logits, topi_ref, gates_ref, rank_ref, imp_ref, cnt_ref,
                    base_ref)


def _combine_gated(x_ref, c0_ref, c1_ref, c2_ref, c3_ref, g_ref):
    g = g_ref[...]
    out = g[:, 0:1] * c0_ref[0]
    out = out + g[:, 1:2] * c1_ref[0]
    out = out + g[:, 2:3] * c2_ref[0]
    out = out + g[:, 3:4] * c3_ref[0]
    return x_ref[...] + out


def _router_body(x_ref, c0_ref, c1_ref, c2_ref, c3_ref, g_ref, Wr_ref, br_ref,
                 x1_ref, topi_ref, gates_ref, rank_ref, imp_ref, cnt_ref, base_ref):
    x = _combine_gated(x_ref, c0_ref, c1_ref, c2_ref, c3_ref, g_ref)
    logits = _dot(x, Wr_ref[...]) + br_ref[...]
    x1_ref[...] = x
    _router_outputs(x, logits, topi_ref, gates_ref, rank_ref, imp_ref, cnt_ref,
                    base_ref)


def _head_body(x_ref, c0_ref, c1_ref, c2_ref, c3_ref, g_ref,
               H1_ref, bh1_ref, H2_ref, bh2_ref, res_ref):
    x = _combine_gated(x_ref, c0_ref, c1_ref, c2_ref, c3_ref, g_ref)
    h = jnp.maximum(_dot(x, H1_ref[...]) + bh1_ref[...], 0.0)
    z = _dot(h, H2_ref[...]) + bh2_ref[...]
    res_ref[...] = 1.0 / (1.0 + jnp.exp(-z))


def _ffn_body(te_ref, xs_ref, W1_ref, b1_ref, W2_ref, b2_ref, ys_ref):
    x = xs_ref[...]
    h = jnp.maximum(_dot(x, W1_ref[0]) + b1_ref[0], 0.0)
    ys_ref[...] = _dot(h, W2_ref[0]) + b2_ref[0]


def _ffn_call(texp, xs, W1l, b1l, W2l, b2l):
    grid_spec = pltpu.PrefetchScalarGridSpec(
        num_scalar_prefetch=1,
        grid=(NTILES,),
        in_specs=[
            pl.BlockSpec((TILE, D), lambda i, te: (i, 0)),
            pl.BlockSpec((1, D, DFF), lambda i, te: (te[i], 0, 0)),
            pl.BlockSpec((1, 1, DFF), lambda i, te: (te[i], 0, 0)),
            pl.BlockSpec((1, DFF, D), lambda i, te: (te[i], 0, 0)),
            pl.BlockSpec((1, 1, D), lambda i, te: (te[i], 0, 0)),
        ],
        out_specs=pl.BlockSpec((TILE, D), lambda i, te: (i, 0)),
    )
    return pl.pallas_call(
        _ffn_body,
        grid_spec=grid_spec,
        out_shape=jax.ShapeDtypeStruct((NPAD, D), jnp.float32),
        interpret=_INTERPRET,
    )(texp, xs, W1l, b1l.reshape(E, 1, DFF), W2l, b2l.reshape(E, 1, D))


def _slots_body(topi_ref, rank_ref, base_ref, slot_ref, texp_ref):
    base_row = base_ref[0, 0, :][None, :]
    iota = jax.lax.broadcasted_iota(jnp.int32, (B, E), 1)
    cols = []
    for k in range(K):
        tk = topi_ref[:, k:k + 1]
        sel = jnp.where(iota == tk, base_row, 0.0)
        cols.append(jnp.sum(sel, axis=1, keepdims=True).astype(jnp.int32))
    slot_ref[...] = jnp.concatenate(cols, axis=1) + rank_ref[...]
    ts = (base_row * (1.0 / TILE)).astype(jnp.int32)
    jv = jax.lax.broadcasted_iota(jnp.int32, (NTILES, E), 0)
    texp_ref[...] = (jnp.sum((jv >= ts).astype(jnp.int32), axis=1, keepdims=True)
                     - 1)


def _slots_call(topi, rank, bas):
    return pl.pallas_call(
        _slots_body,
        in_specs=[pl.BlockSpec((B, K), lambda: (0, 0)),
                  pl.BlockSpec((B, K), lambda: (0, 0)),
                  pl.BlockSpec((1, 1, E), lambda: (0, 0, 0))],
        out_specs=[pl.BlockSpec((B, K), lambda: (0, 0)),
                   pl.BlockSpec((NTILES, 1), lambda: (0, 0))],
        out_shape=[jax.ShapeDtypeStruct((B, K), jnp.int32),
                   jax.ShapeDtypeStruct((NTILES, 1), jnp.int32)],
        interpret=_INTERPRET,
    )(topi, rank, bas)


def _dispatch_body(slott_hbm, x_hbm,
                   xs_hbm,
                   slot_v, rows_v, sem, sem2):
    wid = lax.axis_index("s") * 2 + lax.axis_index("c")
    tokbase = wid * TPW
    cp = pltpu.async_copy(x_hbm.at[pl.ds(tokbase, TPW)], rows_v, sem)
    for k in range(K):
        pltpu.sync_copy(slott_hbm.at[pl.ds(k * B + tokbase, TPW)], slot_v.at[k])
    cp.wait()
    handles = [pltpu.async_copy(rows_v, xs_hbm.at[slot_v.at[k]], sem2)
               for k in range(K)]
    for h in handles:
        h.wait()


def _combine_body(slott_hbm, ys_hbm,
                  c_hbm,
                  slot_v, bufa, bufb, semga, semgb, semwa, semwb):
    wid = lax.axis_index("s") * 2 + lax.axis_index("c")
    tokbase = wid * TPW
    for k in range(K):
        pltpu.sync_copy(slott_hbm.at[pl.ds(k * B + tokbase, TPW)], slot_v.at[k])
    g0 = pltpu.async_copy(ys_hbm.at[slot_v.at[0]], bufa, semga)
    g1 = pltpu.async_copy(ys_hbm.at[slot_v.at[1]], bufb, semgb)
    g0.wait()
    w0 = pltpu.async_copy(bufa, c_hbm.at[0, pl.ds(tokbase, TPW)], semwa)
    g1.wait()
    w1 = pltpu.async_copy(bufb, c_hbm.at[1, pl.ds(tokbase, TPW)], semwb)
    w0.wait()
    g2 = pltpu.async_copy(ys_hbm.at[slot_v.at[2]], bufa, semga)
    w1.wait()
    g3 = pltpu.async_copy(ys_hbm.at[slot_v.at[3]], bufb, semgb)
    g2.wait()
    w2 = pltpu.async_copy(bufa, c_hbm.at[2, pl.ds(tokbase, TPW)], semwa)
    g3.wait()
    w3 = pltpu.async_copy(bufb, c_hbm.at[3, pl.ds(tokbase, TPW)], semwb)
    w2.wait()
    w3.wait()


def _sc_mesh():
    return plsc.VectorSubcoreMesh(core_axis_name="c", subcore_axis_name="s")


def _sc_dispatch(slott_flat, x):
    fn = functools.partial(
        pl.kernel,
        out_type=[jax.ShapeDtypeStruct((NPAD, D), jnp.float32)],
        mesh=_sc_mesh(),
        scratch_types=[
            pltpu.VMEM((K, TPW), jnp.int32),
            pltpu.VMEM((TPW, D), jnp.float32),
            pltpu.SemaphoreType.DMA,
            pltpu.SemaphoreType.DMA,
        ],
    )(_dispatch_body)
    (xs,) = fn(slott_flat, x)
    return xs


def _sc_combine(slott_flat, ys):
    fn = functools.partial(
        pl.kernel,
        out_type=[jax.ShapeDtypeStruct((K, B, D), jnp.float32)],
        mesh=_sc_mesh(),
        scratch_types=[
            pltpu.VMEM((K, TPW), jnp.int32),
            pltpu.VMEM((TPW, D), jnp.float32),
            pltpu.VMEM((TPW, D), jnp.float32),
            pltpu.SemaphoreType.DMA,
            pltpu.SemaphoreType.DMA,
            pltpu.SemaphoreType.DMA,
            pltpu.SemaphoreType.DMA,
        ],
    )(_combine_body)
    (c,) = fn(slott_flat, ys)
    return c


def _full(shape):
    nd = len(shape)
    return pl.BlockSpec(shape, lambda i: (0,) * nd)


_TOKSPEC = lambda: pl.BlockSpec((BLK, D), lambda i: (i, 0))
_K4SPEC_I = lambda: pl.BlockSpec((BLK, K), lambda i: (i, 0))
_ACCSPEC = lambda: pl.BlockSpec((1, 1, E), lambda i: (0, 0, 0))


def _router_outs():
    acc = jax.ShapeDtypeStruct((1, 1, E), jnp.float32)
    return [jax.ShapeDtypeStruct((B, D), jnp.float32),
            jax.ShapeDtypeStruct((B, K), jnp.int32),
            jax.ShapeDtypeStruct((B, K), jnp.float32),
            jax.ShapeDtypeStruct((B, K), jnp.int32),
            acc, acc, acc]


def _router_outspecs():
    return [_TOKSPEC(), _K4SPEC_I(), _K4SPEC_I(), _K4SPEC_I(),
            _ACCSPEC(), _ACCSPEC(), _ACCSPEC()]


def kernel(op_idx, a, b, c, op_table, Wp, bp, Wr, br, W1, b1, W2, b2, H1, bh1, H2, bh2):
    tok3 = lambda v: v.reshape(NBLK, 1, BLK)
    tokspec = pl.BlockSpec((1, 1, BLK), lambda i: (i, 0, 0))

    x0, topi0, gates0, rank0, imp0, cnt0, bas0 = pl.pallas_call(
        _embed_body,
        grid=(NBLK,),
        in_specs=[tokspec, tokspec, tokspec, tokspec,
                  _full((8, 32)), _full((128, D)), _full((1, D)),
                  _full((D, E)), _full((1, E))],
        out_specs=_router_outspecs(),
        out_shape=_router_outs(),
        interpret=_INTERPRET,
    )(tok3(op_idx), tok3(a), tok3(b), tok3(c), op_table,
      jnp.pad(Wp, ((0, 128 - 49), (0, 0))), bp[None, :], Wr[0], br[0][None, :])

    x_in, topi_l, imps, cnts = x0, topi0, [], []
    gates_l, rank_l, bas_l = gates0, rank0, bas0
    imps.append(imp0)
    cnts.append(cnt0)

    for l in range(2):
        slot, texp2 = _slots_call(topi_l, rank_l, bas_l)
        slott_flat = slot.T.reshape(-1)
        texp = texp2.reshape(-1)
        xs = _sc_dispatch(slott_flat, x_in)
        ys = _ffn_call(texp, xs, W1[l], b1[l], W2[l], b2[l])
        c = _sc_combine(slott_flat, ys)
        cin = [c, c, c, c, gates_l]
        cspecs = [pl.BlockSpec((1, BLK, D), lambda i, k=k: (k, i, 0))
                  for k in range(K)] + [_K4SPEC_I()]
        if l == 0:
            x1, topi1, gates1, rank1, imp1, cnt1, bas1 = pl.pallas_call(
                _router_body,
                grid=(NBLK,),
                in_specs=[_TOKSPEC()] + cspecs + [_full((D, E)), _full((1, E))],
                out_specs=_router_outspecs(),
                out_shape=_router_outs(),
                interpret=_INTERPRET,
            )(x_in, *cin, Wr[1], br[1][None, :])
            x_in, topi_l, gates_l, rank_l, bas_l = x1, topi1, gates1, rank1, bas1
            imps.append(imp1)
            cnts.append(cnt1)
        else:
            res = pl.pallas_call(
                _head_body,
                grid=(NBLK,),
                in_specs=[_TOKSPEC()] + cspecs +
                         [_full((D, 64)), _full((1, 64)),
                          _full((64, 8)), _full((1, 8))],
                out_specs=pl.BlockSpec((BLK, 8), lambda i: (i, 0)),
                out_shape=jax.ShapeDtypeStruct((B, 8), jnp.float32),
                interpret=_INTERPRET,
            )(x_in, *cin, H1, bh1[None, :], H2, bh2[None, :])

    inv_b = 1.0 / B
    aux = jnp.float32(0.0)
    for l in range(2):
        aux = aux + E * jnp.sum(imps[l][0, 0] * inv_b * (cnts[l][0, 0] * inv_b))
    return (res, topi_l, aux)


# speedup vs baseline: 1.6592x; 1.0755x over previous
import functools

import jax
import jax.numpy as jnp
from jax import lax
from jax.experimental import pallas as pl
from jax.experimental.pallas import tpu as pltpu
from jax.experimental.pallas import tpu_sc as plsc

B = 4096
D = 256
E = 16
K = 4
DFF = 512
BLK = 512
NBLK = B // BLK

TILE = 256
NPAD = B * K + E * TILE
NTILES = NPAD // TILE
NW = 32
APW = (B * K) // NW
TPW = B // NW

_INTERPRET = False


def _dot(a, b):
    return jnp.dot(a, b, preferred_element_type=jnp.float32)


def _topk_gates(logits):
    l = logits
    iota = jax.lax.broadcasted_iota(jnp.int32, l.shape, 1)
    tvs, tis = [], []
    for _ in range(K):
        m = jnp.max(l, axis=1, keepdims=True)
        idx = jnp.min(jnp.where(l == m, iota, E), axis=1, keepdims=True)
        tvs.append(m)
        tis.append(idx)
        l = jnp.where(iota == idx, -jnp.inf, l)
    topv = jnp.concatenate(tvs, axis=1)
    topi = jnp.concatenate(tis, axis=1)
    g = jnp.exp(topv - topv[:, 0:1])
    gates = g / jnp.sum(g, axis=1, keepdims=True)
    comb = jnp.zeros_like(logits)
    ind = jnp.zeros_like(logits)
    for k in range(K):
        sel = iota == tis[k]
        comb = comb + jnp.where(sel, gates[:, k:k + 1], 0.0)
        ind = ind + jnp.where(sel, 1.0, 0.0)
    return topi, tis, gates, comb, ind


def _ranks_full(tis, ind):
    r_i = jax.lax.broadcasted_iota(jnp.int32, (128, 128), 0)
    c_i = jax.lax.broadcasted_iota(jnp.int32, (128, 128), 1)
    Lx = (r_i > c_i).astype(jnp.float32)
    parts = []
    off = jnp.zeros((1, E), jnp.float32)
    for bk in range(B // 128):
        Mb = ind[bk * 128:(bk + 1) * 128]
        parts.append(_dot(Lx, Mb) + off)
        off = off + jnp.sum(Mb, axis=0, keepdims=True)
    Rm = jnp.concatenate(parts, axis=0)
    iota = jax.lax.broadcasted_iota(jnp.int32, (B, E), 1)
    rks = []
    for k in range(K):
        sel = (iota == tis[k]).astype(jnp.float32)
        rks.append(jnp.sum(sel * Rm, axis=1, keepdims=True))
    return jnp.concatenate(rks, axis=1).astype(jnp.int32), off


def _router_core(logits, topi_ref, gates_ref, slot_ref, texp_ref,
                 imp_ref, cnt_ref):
    topi, tis, gates, _, ind = _topk_gates(logits)
    mx = jnp.max(logits, axis=1, keepdims=True)
    ex = jnp.exp(logits - mx)
    sm = ex / jnp.sum(ex, axis=1, keepdims=True)
    imp_ref[0, 0, :] = jnp.sum(sm, axis=0)
    rank, cnt = _ranks_full(tis, ind)
    cnt_ref[0, 0, :] = cnt[0]
    padded = jnp.floor((cnt + (TILE - 1)) * (1.0 / TILE)) * TILE
    r_e = jax.lax.broadcasted_iota(jnp.int32, (E, E), 0)
    c_e = jax.lax.broadcasted_iota(jnp.int32, (E, E), 1)
    base_row = jnp.sum(jnp.where(r_e > c_e, padded, 0.0), axis=1)[None, :]
    iota = jax.lax.broadcasted_iota(jnp.int32, (B, E), 1)
    cols = []
    for k in range(K):
        sel = iota == tis[k]
        cols.append(jnp.sum(jnp.where(sel, base_row, 0.0), axis=1,
                            keepdims=True).astype(jnp.int32))
    slot_ref[...] = jnp.concatenate(cols, axis=1) + rank
    ts = (base_row * (1.0 / TILE)).astype(jnp.int32)
    jv = jax.lax.broadcasted_iota(jnp.int32, (NTILES, E), 0)
    texp_ref[...] = (jnp.sum((jv >= ts).astype(jnp.int32), axis=1,
                             keepdims=True) - 1)
    topi_ref[...] = topi
    gates_ref[...] = gates


def _embed_body(opi_ref, a_ref, b_ref, c_ref, opt_ref, Wp_ref, bp_ref,
                Wr_ref, br_ref,
                x_ref, topi_ref, gates_ref, slot_ref, texp_ref, imp_ref, cnt_ref):
    op = opi_ref[0, 0, :][:, None]
    av = a_ref[0, 0, :][:, None]
    bv = b_ref[0, 0, :][:, None]
    cv = c_ref[0, 0, :][:, None]
    i8 = jax.lax.broadcasted_iota(jnp.int32, (B, 8), 1)
    abits = ((av >> i8) & 1).astype(jnp.float32)
    bbits = ((bv >> i8) & 1).astype(jnp.float32)
    cf = cv.astype(jnp.float32)
    op_emb = jnp.zeros((B, 32), jnp.float32)
    for j in range(8):
        op_emb = jnp.where(op == j, opt_ref[j:j + 1, :], op_emb)
    feat = jnp.concatenate(
        [op_emb, abits, bbits, cf, jnp.zeros((B, 128 - 49), jnp.float32)], axis=1)
    x = _dot(feat, Wp_ref[...]) + bp_ref[...]
    logits = _dot(x, Wr_ref[...]) + br_ref[...]
    x_ref[...] = x
    _router_core(logits, topi_ref, gates_ref, slot_ref, texp_ref, imp_ref, cnt_ref)


def _combine_gated(x_ref, c0_ref, c1_ref, c2_ref, c3_ref, g_ref):
    g = g_ref[...]
    out = g[:, 0:1] * c0_ref[0]
    out = out + g[:, 1:2] * c1_ref[0]
    out = out + g[:, 2:3] * c2_ref[0]
    out = out + g[:, 3:4] * c3_ref[0]
    return x_ref[...] + out


def _router_body(x_ref, c0_ref, c1_ref, c2_ref, c3_ref, g_ref, Wr_ref, br_ref,
                 x1_ref, topi_ref, gates_ref, slot_ref, texp_ref, imp_ref, cnt_ref):
    x = _combine_gated(x_ref, c0_ref, c1_ref, c2_ref, c3_ref, g_ref)
    logits = _dot(x, Wr_ref[...]) + br_ref[...]
    x1_ref[...] = x
    _router_core(logits, topi_ref, gates_ref, slot_ref, texp_ref, imp_ref, cnt_ref)


def _head_body(x_ref, c0_ref, c1_ref, c2_ref, c3_ref, g_ref,
               H1_ref, bh1_ref, H2_ref, bh2_ref, res_ref):
    x = _combine_gated(x_ref, c0_ref, c1_ref, c2_ref, c3_ref, g_ref)
    h = jnp.maximum(_dot(x, H1_ref[...]) + bh1_ref[...], 0.0)
    z = _dot(h, H2_ref[...]) + bh2_ref[...]
    res_ref[...] = 1.0 / (1.0 + jnp.exp(-z))


def _ffn_body(te_ref, xs_ref, W1_ref, b1_ref, W2_ref, b2_ref, ys_ref):
    x = xs_ref[...]
    h = jnp.maximum(_dot(x, W1_ref[0]) + b1_ref[0], 0.0)
    ys_ref[...] = _dot(h, W2_ref[0]) + b2_ref[0]


def _ffn_call(texp, xs, W1l, b1l, W2l, b2l):
    grid_spec = pltpu.PrefetchScalarGridSpec(
        num_scalar_prefetch=1,
        grid=(NTILES,),
        in_specs=[
            pl.BlockSpec((TILE, D), lambda i, te: (i, 0)),
            pl.BlockSpec((1, D, DFF), lambda i, te: (te[i], 0, 0)),
            pl.BlockSpec((1, 1, DFF), lambda i, te: (te[i], 0, 0)),
            pl.BlockSpec((1, DFF, D), lambda i, te: (te[i], 0, 0)),
            pl.BlockSpec((1, 1, D), lambda i, te: (te[i], 0, 0)),
        ],
        out_specs=pl.BlockSpec((TILE, D), lambda i, te: (i, 0)),
    )
    return pl.pallas_call(
        _ffn_body,
        grid_spec=grid_spec,
        out_shape=jax.ShapeDtypeStruct((NPAD, D), jnp.float32),
        interpret=_INTERPRET,
    )(texp, xs, W1l, b1l.reshape(E, 1, DFF), W2l, b2l.reshape(E, 1, D))


def _dispatch_body(slott_hbm, x_hbm,
                   xs_hbm,
                   slot_v, rows_v, sem, sem2):
    wid = lax.axis_index("s") * 2 + lax.axis_index("c")
    tokbase = wid * TPW
    cp = pltpu.async_copy(x_hbm.at[pl.ds(tokbase, TPW)], rows_v, sem)
    for k in range(K):
        pltpu.sync_copy(slott_hbm.at[pl.ds(k * B + tokbase, TPW)], slot_v.at[k])
    cp.wait()
    handles = [pltpu.async_copy(rows_v, xs_hbm.at[slot_v.at[k]], sem2)
               for k in range(K)]
    for h in handles:
        h.wait()


def _combine_body(slott_hbm, ys_hbm,
                  c_hbm,
                  slot_v, bufa, bufb, semga, semgb, semwa, semwb):
    wid = lax.axis_index("s") * 2 + lax.axis_index("c")
    tokbase = wid * TPW
    for k in range(K):
        pltpu.sync_copy(slott_hbm.at[pl.ds(k * B + tokbase, TPW)], slot_v.at[k])
    g0 = pltpu.async_copy(ys_hbm.at[slot_v.at[0]], bufa, semga)
    g1 = pltpu.async_copy(ys_hbm.at[slot_v.at[1]], bufb, semgb)
    g0.wait()
    w0 = pltpu.async_copy(bufa, c_hbm.at[0, pl.ds(tokbase, TPW)], semwa)
    g1.wait()
    w1 = pltpu.async_copy(bufb, c_hbm.at[1, pl.ds(tokbase, TPW)], semwb)
    w0.wait()
    g2 = pltpu.async_copy(ys_hbm.at[slot_v.at[2]], bufa, semga)
    w1.wait()
    g3 = pltpu.async_copy(ys_hbm.at[slot_v.at[3]], bufb, semgb)
    g2.wait()
    w2 = pltpu.async_copy(bufa, c_hbm.at[2, pl.ds(tokbase, TPW)], semwa)
    g3.wait()
    w3 = pltpu.async_copy(bufb, c_hbm.at[3, pl.ds(tokbase, TPW)], semwb)
    w2.wait()
    w3.wait()


def _sc_mesh():
    return plsc.VectorSubcoreMesh(core_axis_name="c", subcore_axis_name="s")


def _sc_dispatch(slott_flat, x):
    fn = functools.partial(
        pl.kernel,
        out_type=[jax.ShapeDtypeStruct((NPAD, D), jnp.float32)],
        mesh=_sc_mesh(),
        scratch_types=[
            pltpu.VMEM((K, TPW), jnp.int32),
            pltpu.VMEM((TPW, D), jnp.float32),
            pltpu.SemaphoreType.DMA,
            pltpu.SemaphoreType.DMA,
        ],
    )(_dispatch_body)
    (xs,) = fn(slott_flat, x)
    return xs


def _sc_combine(slott_flat, ys):
    fn = functools.partial(
        pl.kernel,
        out_type=[jax.ShapeDtypeStruct((K, B, D), jnp.float32)],
        mesh=_sc_mesh(),
        scratch_types=[
            pltpu.VMEM((K, TPW), jnp.int32),
            pltpu.VMEM((TPW, D), jnp.float32),
            pltpu.VMEM((TPW, D), jnp.float32),
            pltpu.SemaphoreType.DMA,
            pltpu.SemaphoreType.DMA,
            pltpu.SemaphoreType.DMA,
            pltpu.SemaphoreType.DMA,
        ],
    )(_combine_body)
    (c,) = fn(slott_flat, ys)
    return c


def _full(shape):
    nd = len(shape)
    return pl.BlockSpec(shape, lambda *a: (0,) * nd)


_TOKSPEC = lambda: pl.BlockSpec((BLK, D), lambda i: (i, 0))
_K4SPEC_I = lambda: pl.BlockSpec((BLK, K), lambda i: (i, 0))


def _router_outs():
    acc = jax.ShapeDtypeStruct((1, 1, E), jnp.float32)
    return [jax.ShapeDtypeStruct((B, D), jnp.float32),
            jax.ShapeDtypeStruct((B, K), jnp.int32),
            jax.ShapeDtypeStruct((B, K), jnp.float32),
            jax.ShapeDtypeStruct((B, K), jnp.int32),
            jax.ShapeDtypeStruct((NTILES, 1), jnp.int32),
            acc, acc]


def _router_outspecs():
    return [_full((B, D)), _full((B, K)), _full((B, K)), _full((B, K)),
            _full((NTILES, 1)), _full((1, 1, E)), _full((1, 1, E))]


def kernel(op_idx, a, b, c, op_table, Wp, bp, Wr, br, W1, b1, W2, b2, H1, bh1, H2, bh2):
    tok3 = lambda v: v.reshape(1, 1, B)

    x0, topi0, gates0, slot0, texp0, imp0, cnt0 = pl.pallas_call(
        _embed_body,
        grid=(1,),
        in_specs=[_full((1, 1, B))] * 4 +
                 [_full((8, 32)), _full((128, D)), _full((1, D)),
                  _full((D, E)), _full((1, E))],
        out_specs=_router_outspecs(),
        out_shape=_router_outs(),
        interpret=_INTERPRET,
    )(tok3(op_idx), tok3(a), tok3(b), tok3(c), op_table,
      jnp.pad(Wp, ((0, 128 - 49), (0, 0))), bp[None, :], Wr[0], br[0][None, :])

    x_in, topi_l, imps, cnts = x0, topi0, [], []
    gates_l, slot_l, texp_l = gates0, slot0, texp0
    imps.append(imp0)
    cnts.append(cnt0)

    for l in range(2):
        slott_flat = slot_l.T.reshape(-1)
        texp = texp_l.reshape(-1)
        xs = _sc_dispatch(slott_flat, x_in)
        ys = _ffn_call(texp, xs, W1[l], b1[l], W2[l], b2[l])
        cc = _sc_combine(slott_flat, ys)
        if l == 0:
            cspecs = [pl.BlockSpec((1, B, D), lambda i, k=k: (k, 0, 0))
                      for k in range(K)] + [_full((B, K))]
            x1, topi1, gates1, slot1, texp1, imp1, cnt1 = pl.pallas_call(
                _router_body,
                grid=(1,),
                in_specs=[_full((B, D))] + cspecs +
                         [_full((D, E)), _full((1, E))],
                out_specs=_router_outspecs(),
                out_shape=_router_outs(),
                interpret=_INTERPRET,
            )(x_in, cc, cc, cc, cc, gates_l, Wr[1], br[1][None, :])
            x_in, topi_l, gates_l, slot_l, texp_l = x1, topi1, gates1, slot1, texp1
            imps.append(imp1)
            cnts.append(cnt1)
        else:
            cspecs = [pl.BlockSpec((1, BLK, D), lambda i, k=k: (k, i, 0))
                      for k in range(K)] + [_K4SPEC_I()]
            res = pl.pallas_call(
                _head_body,
                grid=(NBLK,),
                in_specs=[_TOKSPEC()] + cspecs +
                         [_full((D, 64)), _full((1, 64)),
                          _full((64, 8)), _full((1, 8))],
                out_specs=pl.BlockSpec((BLK, 8), lambda i: (i, 0)),
                out_shape=jax.ShapeDtypeStruct((B, 8), jnp.float32),
                interpret=_INTERPRET,
            )(x_in, cc, cc, cc, cc, gates_l, H1, bh1[None, :], H2, bh2[None, :])

    inv_b = 1.0 / B
    aux = jnp.float32(0.0)
    for l in range(2):
        aux = aux + E * jnp.sum(imps[l][0, 0] * inv_b * (cnts[l][0, 0] * inv_b))
    return (res, topi_l, aux)
